# baseline (speedup 1.0000x reference)
import jax
import jax.numpy as jnp
from jax import lax
from jax.experimental import pallas as pl

N_SLOTS_K = 8
EPS = 1e-12


def _greedy_body(features_ref, out_ref):
    f = features_ref[0]
    n, d = f.shape
    sal2 = jnp.sum(f * f, axis=1)
    sal_row = jnp.sqrt(sal2).reshape(1, n)
    denom_row = jnp.maximum(sal_row, EPS)
    iota_row = lax.broadcasted_iota(jnp.int32, (1, n), 1)

    mask_row = jnp.ones((1, n), dtype=f.dtype)
    for r in range(N_SLOTS_K):
        ms = sal_row * mask_row
        mx = jnp.max(ms)
        idx = jnp.min(jnp.where(ms == mx, iota_row, n)).astype(jnp.int32)
        sel = features_ref[0, pl.ds(idx, 1), :]
        out_ref[0, pl.ds(r, 1), :] = sel
        dots = lax.dot_general(
            sel, f, (((1,), (1,)), ((), ())),
            preferred_element_type=jnp.float32,
            precision=lax.Precision.HIGHEST,
        )
        snorm = jnp.maximum(jnp.sqrt(jnp.sum(sel * sel)), EPS)
        sim = dots / (denom_row * snorm)
        mask_row = mask_row * (1.0 - jnp.clip(sim, 0.0, 1.0))


def kernel(batch_size, features, fallback):
    del batch_size, fallback
    b, n, d = features.shape
    return pl.pallas_call(
        _greedy_body,
        grid=(b,),
        in_specs=[pl.BlockSpec((1, n, d), lambda i: (i, 0, 0))],
        out_specs=pl.BlockSpec((1, N_SLOTS_K, d), lambda i: (i, 0, 0)),
        out_shape=jax.ShapeDtypeStruct((b, N_SLOTS_K, d), features.dtype),
    )(features)


# speedup vs baseline: 2.7252x; 2.7252x over previous
import jax
import jax.numpy as jnp
from jax import lax
from jax.experimental import pallas as pl

N_SLOTS_K = 8
EPS = 1e-12
BB = 8


def _greedy_body(features_ref, out_ref):
    _, n, d = features_ref.shape
    iota_row = lax.broadcasted_iota(jnp.int32, (1, n), 1)
    for b in range(BB):
        fb = features_ref[b]
        sal_row = jnp.sqrt(jnp.sum(fb * fb, axis=1)).reshape(1, n)
        denom_row = jnp.maximum(sal_row, EPS)
        mask_row = jnp.ones((1, n), dtype=jnp.float32)
        for r in range(N_SLOTS_K):
            ms = sal_row * mask_row
            mx = jnp.max(ms)
            idx = jnp.min(jnp.where(ms == mx, iota_row, n)).astype(jnp.int32)
            sel = features_ref[b, pl.ds(idx, 1), :]
            out_ref[b, pl.ds(r, 1), :] = sel
            dots = jnp.sum(features_ref[b] * sel, axis=1).reshape(1, n)
            snorm = jnp.maximum(jnp.sqrt(jnp.sum(sel * sel)), EPS)
            sim = dots / (denom_row * snorm)
            mask_row = mask_row * (1.0 - jnp.clip(sim, 0.0, 1.0))


def kernel(batch_size, features, fallback):
    del batch_size, fallback
    b, n, d = features.shape
    return pl.pallas_call(
        _greedy_body,
        grid=(b // BB,),
        in_specs=[pl.BlockSpec((BB, n, d), lambda i: (i, 0, 0))],
        out_specs=pl.BlockSpec((BB, N_SLOTS_K, d), lambda i: (i, 0, 0)),
        out_shape=jax.ShapeDtypeStruct((b, N_SLOTS_K, d), features.dtype),
    )(features)
